# Initial kernel scaffold; baseline (speedup 1.0000x reference)
import jax
import jax.numpy as jnp
from jax.experimental import pallas as pl


def kernel(dense_inputs, sparse_inputs, tables, W0, b0, W1, b1, W2, b2):
    raise NotImplementedError("write your pallas kernel here")


# speedup vs baseline: 2.0156x; 2.0156x over previous
import functools

import jax
import jax.numpy as jnp
from jax import lax
from jax.experimental import pallas as pl
from jax.experimental.pallas import tpu as pltpu
from jax.experimental.pallas import tpu_sc as plsc

B = 16384
NUM_DENSE = 13
NF = 26
VOCAB = 100000
EMB = 32
R = B * NF

NC, NS = 2, 16
NW = NC * NS
RW = R // NW
IDX_ROWS = RW // 128
SUB_PER_CHUNK = 8
CHUNK = SUB_PER_CHUNK * 128
NCHUNK = RW // CHUNK


def _sc_gather(sparse2d, tables_flat):
    mesh = plsc.VectorSubcoreMesh(core_axis_name="c", subcore_axis_name="s")

    @functools.partial(
        pl.kernel,
        out_type=jax.ShapeDtypeStruct((R, EMB), jnp.float32),
        mesh=mesh,
        compiler_params=pltpu.CompilerParams(use_tc_tiling_on_sc=False),
        scratch_types=[
            pltpu.VMEM((IDX_ROWS, 128), jnp.int32),
            pltpu.VMEM((CHUNK, EMB), jnp.float32),
            pltpu.SemaphoreType.DMA,
        ],
    )
    def k(sparse_hbm, tables_hbm, out_hbm, idx_v, rows_v, sem):
        wid = lax.axis_index("s") * NC + lax.axis_index("c")
        base = wid * RW

        pltpu.sync_copy(sparse_hbm.at[pl.ds(wid * IDX_ROWS, IDX_ROWS)], idx_v)

        def row_body(i, carry):
            for j in range(128 // 16):
                g0 = base + i * 128 + j * 16
                lanes = g0 + lax.iota(jnp.int32, 16)
                off = (lanes % NF) * VOCAB
                sl = pl.ds(j * 16, 16)
                idx_v[i, sl] = idx_v[i, sl] + off
            return carry

        lax.fori_loop(0, IDX_ROWS, row_body, 0)

        def chunk_body(c, carry):
            handles = []
            for s in range(SUB_PER_CHUNK):
                h = pltpu.async_copy(
                    tables_hbm.at[idx_v.at[c * SUB_PER_CHUNK + s]],
                    rows_v.at[pl.ds(s * 128, 128)],
                    sem,
                )
                handles.append(h)
            for h in handles:
                h.wait()
            pltpu.sync_copy(rows_v, out_hbm.at[pl.ds(base + c * CHUNK, CHUNK)])
            return carry

        lax.fori_loop(0, NCHUNK, chunk_body, 0)

    return k(sparse2d, tables_flat)


def _mlp(embed, dense_pad, w0e, w0d, b0, w1, b1, w2, b2, bs=1024):
    h0 = b0.shape[-1]
    h1 = b1.shape[-1]
    h2 = b2.shape[-1]
    de = embed.shape[-1]
    dd = dense_pad.shape[-1]

    def body(e_ref, d_ref, w0e_ref, w0d_ref, b0_ref, w1_ref, b1_ref,
             w2_ref, b2_ref, out_ref):
        x = jnp.dot(e_ref[...], w0e_ref[...], preferred_element_type=jnp.float32)
        x = x + jnp.dot(d_ref[...], w0d_ref[...], preferred_element_type=jnp.float32)
        x = jnp.maximum(x + b0_ref[...], 0.0)
        x = jnp.maximum(
            jnp.dot(x, w1_ref[...], preferred_element_type=jnp.float32) + b1_ref[...], 0.0)
        x = jnp.maximum(
            jnp.dot(x, w2_ref[...], preferred_element_type=jnp.float32) + b2_ref[...], 0.0)
        out_ref[...] = x

    return pl.pallas_call(
        body,
        grid=(B // bs,),
        in_specs=[
            pl.BlockSpec((bs, de), lambda i: (i, 0)),
            pl.BlockSpec((bs, dd), lambda i: (i, 0)),
            pl.BlockSpec((de, h0), lambda i: (0, 0)),
            pl.BlockSpec((dd, h0), lambda i: (0, 0)),
            pl.BlockSpec((1, h0), lambda i: (0, 0)),
            pl.BlockSpec((h0, h1), lambda i: (0, 0)),
            pl.BlockSpec((1, h1), lambda i: (0, 0)),
            pl.BlockSpec((h1, h2), lambda i: (0, 0)),
            pl.BlockSpec((1, h2), lambda i: (0, 0)),
        ],
        out_specs=pl.BlockSpec((bs, h2), lambda i: (i, 0)),
        out_shape=jax.ShapeDtypeStruct((B, h2), jnp.float32),
    )(embed, dense_pad, w0e, w0d, b0.reshape(1, h0), w1, b1.reshape(1, h1),
      w2, b2.reshape(1, h2))


def kernel(dense_inputs, sparse_inputs, tables, W0, b0, W1, b1, W2, b2):
    sparse2d = sparse_inputs.astype(jnp.int32).reshape(R // 128, 128)
    tables_flat = tables.reshape(NF * VOCAB, EMB)
    embed = _sc_gather(sparse2d, tables_flat).reshape(B, NF * EMB)

    dense_pad = jnp.pad(dense_inputs, ((0, 0), (0, 128 - NUM_DENSE)))
    w0d = jnp.pad(W0[:NUM_DENSE], ((0, 128 - NUM_DENSE), (0, 0)))
    w0e = W0[NUM_DENSE:]
    return _mlp(embed, dense_pad, w0e, w0d, b0, W1, b1, W2, b2)


# speedup vs baseline: 4.7196x; 2.3416x over previous
import functools

import jax
import jax.numpy as jnp
from jax import lax
from jax.experimental import pallas as pl
from jax.experimental.pallas import tpu as pltpu
from jax.experimental.pallas import tpu_sc as plsc

B = 16384
NUM_DENSE = 13
NF = 26
VOCAB = 100000
EMB = 32
R = B * NF

VCAP = 106496
TR_CHUNK = 8192
TBL_ROWS = NF * VCAP

NC, NS = 2, 16
NW = NC * NS
RW = R // NW
IDX_ROWS = RW // 128
SUB_PER_CHUNK = 8
CHUNK = SUB_PER_CHUNK * 128
NCHUNK = RW // CHUNK


def _tc_transpose(tables_t):
    q = TR_CHUNK // 4

    def body(i0, i1, i2, i3, out_ref):
        z = jnp.concatenate([i0[0], i1[0], i2[0], i3[0]], axis=0)
        out_ref[...] = jnp.transpose(z, (1, 0))

    last_blk = (VOCAB - 1) // q

    def in_spec(j):
        return pl.BlockSpec(
            (1, EMB, q),
            lambda f, c, j=j: (f, 0, jnp.minimum(4 * c + j, last_blk)))

    return pl.pallas_call(
        body,
        grid=(NF, VCAP // TR_CHUNK),
        in_specs=[in_spec(0), in_spec(1), in_spec(2), in_spec(3)],
        out_specs=pl.BlockSpec((q, 128),
                               lambda f, c: (f * (VCAP // TR_CHUNK) + c, 0)),
        out_shape=jax.ShapeDtypeStruct((TBL_ROWS // 4, 128), jnp.float32),
    )(tables_t, tables_t, tables_t, tables_t)


def _sc_gather(sparse2d, tables_flat):
    mesh = plsc.VectorSubcoreMesh(core_axis_name="c", subcore_axis_name="s")

    @functools.partial(
        pl.kernel,
        out_type=jax.ShapeDtypeStruct((R, EMB), jnp.float32),
        name="sc_embed_gather",
        mesh=mesh,
        compiler_params=pltpu.CompilerParams(use_tc_tiling_on_sc=False),
        scratch_types=[
            pltpu.VMEM((IDX_ROWS, 128), jnp.int32),
            pltpu.VMEM((CHUNK, EMB), jnp.float32),
            pltpu.SemaphoreType.DMA,
        ],
    )
    def k(sparse_hbm, tables_hbm, out_hbm, idx_v, rows_v, sem):
        wid = lax.axis_index("s") * NC + lax.axis_index("c")
        base = wid * RW

        pltpu.sync_copy(sparse_hbm.at[pl.ds(wid * IDX_ROWS, IDX_ROWS)], idx_v)

        def row_body(i, carry):
            for j in range(128 // 16):
                g0 = base + i * 128 + j * 16
                lanes = g0 + lax.iota(jnp.int32, 16)
                off = (lanes % NF) * VCAP
                sl = pl.ds(j * 16, 16)
                v = idx_v[i, sl]
                p = (((v >> 13) << 13) + ((v & 2047) << 2) + ((v >> 11) & 3))
                idx_v[i, sl] = off + p
            return carry

        lax.fori_loop(0, IDX_ROWS, row_body, 0)

        def chunk_body(c, carry):
            handles = []
            for s in range(SUB_PER_CHUNK):
                h = pltpu.async_copy(
                    tables_hbm.at[idx_v.at[c * SUB_PER_CHUNK + s]],
                    rows_v.at[pl.ds(s * 128, 128)],
                    sem,
                )
                handles.append(h)
            for h in handles:
                h.wait()
            pltpu.sync_copy(rows_v, out_hbm.at[pl.ds(base + c * CHUNK, CHUNK)])
            return carry

        lax.fori_loop(0, NCHUNK, chunk_body, 0)

    return k(sparse2d, tables_flat)


def _mlp(embed, dense_pad, w0e, w0d, b0, w1, b1, w2, b2, bs=1024):
    h0 = b0.shape[-1]
    h1 = b1.shape[-1]
    h2 = b2.shape[-1]
    de = embed.shape[-1]
    dd = dense_pad.shape[-1]

    def body(e_ref, d_ref, w0e_ref, w0d_ref, b0_ref, w1_ref, b1_ref,
             w2_ref, b2_ref, out_ref):
        x = jnp.dot(e_ref[...], w0e_ref[...], preferred_element_type=jnp.float32)
        x = x + jnp.dot(d_ref[...], w0d_ref[...], preferred_element_type=jnp.float32)
        x = jnp.maximum(x + b0_ref[...], 0.0)
        x = jnp.maximum(
            jnp.dot(x, w1_ref[...], preferred_element_type=jnp.float32) + b1_ref[...], 0.0)
        x = jnp.maximum(
            jnp.dot(x, w2_ref[...], preferred_element_type=jnp.float32) + b2_ref[...], 0.0)
        out_ref[...] = x

    return pl.pallas_call(
        body,
        grid=(B // bs,),
        in_specs=[
            pl.BlockSpec((bs, de), lambda i: (i, 0)),
            pl.BlockSpec((bs, dd), lambda i: (i, 0)),
            pl.BlockSpec((de, h0), lambda i: (0, 0)),
            pl.BlockSpec((dd, h0), lambda i: (0, 0)),
            pl.BlockSpec((1, h0), lambda i: (0, 0)),
            pl.BlockSpec((h0, h1), lambda i: (0, 0)),
            pl.BlockSpec((1, h1), lambda i: (0, 0)),
            pl.BlockSpec((h1, h2), lambda i: (0, 0)),
            pl.BlockSpec((1, h2), lambda i: (0, 0)),
        ],
        out_specs=pl.BlockSpec((bs, h2), lambda i: (i, 0)),
        out_shape=jax.ShapeDtypeStruct((B, h2), jnp.float32),
    )(embed, dense_pad, w0e, w0d, b0.reshape(1, h0), w1, b1.reshape(1, h1),
      w2, b2.reshape(1, h2))


def kernel(dense_inputs, sparse_inputs, tables, W0, b0, W1, b1, W2, b2):
    sparse2d = sparse_inputs.astype(jnp.int32).reshape(R // 128, 128)
    tables_t = jnp.transpose(tables, (0, 2, 1))
    tables_flat = _tc_transpose(tables_t).reshape(TBL_ROWS, EMB)
    embed = _sc_gather(sparse2d, tables_flat).reshape(B, NF * EMB)

    dense_pad = jnp.pad(dense_inputs, ((0, 0), (0, 128 - NUM_DENSE)))
    w0d = jnp.pad(W0[:NUM_DENSE], ((0, 128 - NUM_DENSE), (0, 0)))
    w0e = W0[NUM_DENSE:]
    return _mlp(embed, dense_pad, w0e, w0d, b0, W1, b1, W2, b2)


# speedup vs baseline: 5.2250x; 1.1071x over previous
import functools

import jax
import jax.numpy as jnp
from jax import lax
from jax.experimental import pallas as pl
from jax.experimental.pallas import tpu as pltpu
from jax.experimental.pallas import tpu_sc as plsc

B = 16384
NUM_DENSE = 13
NF = 26
VOCAB = 100000
EMB = 32
HIDDEN0 = 256
R = B * NF

VCAP = 106496
TR_CHUNK = 8192
TBL_ROWS = NF * VCAP

NC, NS = 2, 16
NW = NC * NS
RW = R // NW
IDX_ROWS = RW // 128
SUB_PER_CHUNK = 8
CHUNK = SUB_PER_CHUNK * 128
NCHUNK = RW // CHUNK


def _tc_transpose(tables_t):
    q = TR_CHUNK // 8

    def body(*refs):
        out_ref = refs[-1]
        z = jnp.concatenate([r[0] for r in refs[:-1]], axis=0)
        zp = pltpu.bitcast(z.astype(jnp.bfloat16), jnp.float32)
        out_ref[...] = jnp.transpose(zp, (1, 0))

    last_blk = (VOCAB - 1) // q

    def in_spec(j):
        return pl.BlockSpec(
            (1, EMB, q),
            lambda f, c, j=j: (f, 0, jnp.minimum(8 * c + j, last_blk)))

    return pl.pallas_call(
        body,
        grid=(NF, VCAP // TR_CHUNK),
        in_specs=[in_spec(j) for j in range(8)],
        out_specs=pl.BlockSpec((q, 128),
                               lambda f, c: (f * (VCAP // TR_CHUNK) + c, 0)),
        out_shape=jax.ShapeDtypeStruct((TBL_ROWS // 8, 128), jnp.float32),
    )(*([tables_t] * 8))


def _sc_gather(sparse2d, tables_flat):
    mesh = plsc.VectorSubcoreMesh(core_axis_name="c", subcore_axis_name="s")

    @functools.partial(
        pl.kernel,
        out_type=jax.ShapeDtypeStruct((R, 16), jnp.float32),
        name="sc_embed_gather",
        mesh=mesh,
        compiler_params=pltpu.CompilerParams(use_tc_tiling_on_sc=False),
        scratch_types=[
            pltpu.VMEM((IDX_ROWS, 128), jnp.int32),
            pltpu.VMEM((CHUNK, 16), jnp.float32),
            pltpu.SemaphoreType.DMA,
        ],
    )
    def k(sparse_hbm, tables_hbm, out_hbm, idx_v, rows_v, sem):
        wid = lax.axis_index("s") * NC + lax.axis_index("c")
        base = wid * RW

        pltpu.sync_copy(sparse_hbm.at[pl.ds(wid * IDX_ROWS, IDX_ROWS)], idx_v)

        def row_body(i, carry):
            for j in range(128 // 16):
                g0 = base + i * 128 + j * 16
                lanes = g0 + lax.iota(jnp.int32, 16)
                off = (lanes % NF) * VCAP
                sl = pl.ds(j * 16, 16)
                v = idx_v[i, sl]
                p = (((v >> 13) << 13) + ((v & 1023) << 3) + ((v >> 10) & 7))
                idx_v[i, sl] = off + p
            return carry

        lax.fori_loop(0, IDX_ROWS, row_body, 0)

        def chunk_body(c, carry):
            handles = []
            for s in range(SUB_PER_CHUNK):
                h = pltpu.async_copy(
                    tables_hbm.at[idx_v.at[c * SUB_PER_CHUNK + s]],
                    rows_v.at[pl.ds(s * 128, 128)],
                    sem,
                )
                handles.append(h)
            for h in handles:
                h.wait()
            pltpu.sync_copy(rows_v, out_hbm.at[pl.ds(base + c * CHUNK, CHUNK)])
            return carry

        lax.fori_loop(0, NCHUNK, chunk_body, 0)

    return k(sparse2d, tables_flat)


def _mlp(embed_raw, dense_pad, w0ev, w0od, w0d, b0, w1, b1, w2, b2, bs=1024):
    h0 = b0.shape[-1]
    h1 = b1.shape[-1]
    h2 = b2.shape[-1]
    de = embed_raw.shape[-1]
    dd = dense_pad.shape[-1]

    def body(e_ref, d_ref, w0ev_ref, w0od_ref, w0d_ref, b0_ref, w1_ref,
             b1_ref, w2_ref, b2_ref, out_ref):
        u = pltpu.bitcast(e_ref[...], jnp.bfloat16)
        u3 = u.reshape(bs, 2, de)
        ue = u3[:, 0, :]
        uo = u3[:, 1, :]
        x = jnp.dot(ue, w0ev_ref[...], preferred_element_type=jnp.float32)
        x = x + jnp.dot(uo, w0od_ref[...], preferred_element_type=jnp.float32)
        x = x + jnp.dot(d_ref[...], w0d_ref[...], preferred_element_type=jnp.float32)
        x = jnp.maximum(x + b0_ref[...], 0.0)
        x = jnp.maximum(
            jnp.dot(x, w1_ref[...], preferred_element_type=jnp.float32) + b1_ref[...], 0.0)
        x = jnp.maximum(
            jnp.dot(x, w2_ref[...], preferred_element_type=jnp.float32) + b2_ref[...], 0.0)
        out_ref[...] = x

    return pl.pallas_call(
        body,
        grid=(B // bs,),
        in_specs=[
            pl.BlockSpec((bs, de), lambda i: (i, 0)),
            pl.BlockSpec((bs, dd), lambda i: (i, 0)),
            pl.BlockSpec((de, h0), lambda i: (0, 0)),
            pl.BlockSpec((de, h0), lambda i: (0, 0)),
            pl.BlockSpec((dd, h0), lambda i: (0, 0)),
            pl.BlockSpec((1, h0), lambda i: (0, 0)),
            pl.BlockSpec((h0, h1), lambda i: (0, 0)),
            pl.BlockSpec((1, h1), lambda i: (0, 0)),
            pl.BlockSpec((h1, h2), lambda i: (0, 0)),
            pl.BlockSpec((1, h2), lambda i: (0, 0)),
        ],
        out_specs=pl.BlockSpec((bs, h2), lambda i: (i, 0)),
        out_shape=jax.ShapeDtypeStruct((B, h2), jnp.float32),
    )(embed_raw, dense_pad, w0ev, w0od, w0d, b0.reshape(1, h0), w1,
      b1.reshape(1, h1), w2, b2.reshape(1, h2))


def kernel(dense_inputs, sparse_inputs, tables, W0, b0, W1, b1, W2, b2):
    sparse2d = sparse_inputs.astype(jnp.int32).reshape(R // 128, 128)
    tables_t = jnp.transpose(tables, (0, 2, 1))
    tables_packed = _tc_transpose(tables_t).reshape(TBL_ROWS, 16)
    raw = _sc_gather(sparse2d, tables_packed)
    embed_raw = raw.reshape(B, NF * 16)

    dense_pad = jnp.pad(dense_inputs, ((0, 0), (0, 128 - NUM_DENSE)))
    w0d = jnp.pad(W0[:NUM_DENSE], ((0, 128 - NUM_DENSE), (0, 0)))
    w0e3 = W0[NUM_DENSE:].reshape(NF * 16, 2, HIDDEN0)
    w0ev = w0e3[:, 0, :].astype(jnp.bfloat16)
    w0od = w0e3[:, 1, :].astype(jnp.bfloat16)
    return _mlp(embed_raw, dense_pad, w0ev, w0od, w0d, b0, W1, b1, W2, b2)
